# Initial kernel scaffold; baseline (speedup 1.0000x reference)
import jax
import jax.numpy as jnp
from jax.experimental import pallas as pl


def kernel(x, edge_index, batch, W1, b1, W2, b2, W3, b3, W4, b4, W5, b5, Wl, bl):
    raise NotImplementedError("write your pallas kernel here")


# speedup vs baseline: 20.5517x; 20.5517x over previous
import functools

import jax
import jax.numpy as jnp
from jax import lax
from jax.experimental import pallas as pl
from jax.experimental.pallas import tpu as pltpu
from jax.experimental.pallas import tpu_sc as plsc

_NC = 2
_NS = 16
_NW = _NC * _NS
_CH = 128
_F = 16
_G = 64


def _sc_agg_body(u_hbm, src_hbm, dst_hbm, zro_hbm, out_hbm,
                 sidx, didx, rows0, rows1, acc, sem0, sem1, *, cpw, rps):
    c = lax.axis_index("c")
    s = lax.axis_index("s")
    w = c * _NS + s
    pltpu.sync_copy(zro_hbm.at[pl.ds(s * rps, rps)], acc.at[pl.ds(s * rps, rps)])
    plsc.subcore_barrier()
    base = w * cpw

    def body(j, carry):
        i0 = base + 2 * j
        pltpu.sync_copy(src_hbm.at[pl.ds(i0, 2)], sidx)
        pltpu.sync_copy(dst_hbm.at[pl.ds(i0, 2)], didx)
        cp0 = pltpu.async_copy(u_hbm.at[sidx.at[0]], rows0, sem0)
        cp1 = pltpu.async_copy(u_hbm.at[sidx.at[1]], rows1, sem1)
        cp0.wait()
        pltpu.sync_copy(rows0, acc.at[didx.at[0]], add=True)
        cp1.wait()
        pltpu.sync_copy(rows1, acc.at[didx.at[1]], add=True)
        return carry

    lax.fori_loop(0, cpw // 2, body, 0, unroll=False)
    plsc.subcore_barrier()
    pltpu.sync_copy(acc.at[pl.ds(s * rps, rps)], out_hbm.at[c, pl.ds(s * rps, rps)])


def _make_agg(npad, cpw, rps):
    mesh = plsc.VectorSubcoreMesh(core_axis_name="c", subcore_axis_name="s")
    return pl.kernel(
        functools.partial(_sc_agg_body, cpw=cpw, rps=rps),
        out_type=jax.ShapeDtypeStruct((_NC, npad, _F), jnp.float32),
        mesh=mesh,
        scratch_types=[
            pltpu.VMEM((2, _CH), jnp.int32),
            pltpu.VMEM((2, _CH), jnp.int32),
            pltpu.VMEM((_CH, _F), jnp.float32),
            pltpu.VMEM((_CH, _F), jnp.float32),
            pltpu.VMEM_SHARED((npad, _F), jnp.float32),
            pltpu.SemaphoreType.DMA,
            pltpu.SemaphoreType.DMA,
        ],
        compiler_params=pltpu.CompilerParams(use_tc_tiling_on_sc=False),
    )


def _pre_body(xr, wr, dcr, zr, ur, disr, dinvr):
    dc = dcr[...]
    deg = dc[0] + dc[1] + 1.0
    dis = lax.rsqrt(deg)
    dinv = 1.0 / deg
    z = jnp.dot(xr[...], wr[...], preferred_element_type=jnp.float32)
    zr[...] = z
    ur[...] = z * dis
    disr[...] = dis
    dinvr[...] = dinv


def _mid_body(ar, zr, disr, dinvr, br, wr, z2r, u2r):
    a = ar[...]
    dis = disr[...]
    h = jnp.maximum(dis * (a[0] + a[1]) + dinvr[...] * zr[...] + br[...], 0.0)
    z2 = jnp.dot(h, wr[...], preferred_element_type=jnp.float32)
    z2r[...] = z2
    u2r[...] = z2 * dis


def _post_body(ar, zr, disr, dinvr, br, batr, wlr, blr, outr, sacc, cacc, *, nblk):
    i = pl.program_id(0)

    @pl.when(i == 0)
    def _init():
        sacc[...] = jnp.zeros_like(sacc)
        cacc[...] = jnp.zeros_like(cacc)

    a = ar[...]
    h = jnp.maximum(disr[...] * (a[0] + a[1]) + dinvr[...] * zr[...] + br[...], 0.0)
    oh = (batr[...] == lax.broadcasted_iota(jnp.int32, (h.shape[0], _G), 1))
    oh = oh.astype(jnp.float32)
    dn = (((0,), (0,)), ((), ()))
    sacc[...] += lax.dot_general(oh, h, dn, preferred_element_type=jnp.float32)
    ones = jnp.ones((h.shape[0], 1), jnp.float32)
    cacc[...] += lax.dot_general(oh, ones, dn, preferred_element_type=jnp.float32)

    @pl.when(i == nblk - 1)
    def _fin():
        mean = sacc[...] / jnp.maximum(cacc[...], 1.0)
        logits = jnp.dot(mean, wlr[...], preferred_element_type=jnp.float32) + blr[...]
        m = jnp.max(logits, axis=1, keepdims=True)
        lse = jnp.log(jnp.sum(jnp.exp(logits - m), axis=1, keepdims=True)) + m
        outr[...] = logits - lse


def kernel(x, edge_index, batch, W1, b1, W2, b2, W3, b3, W4, b4, W5, b5, Wl, bl):
    n, fin = x.shape
    e = edge_index.shape[1]
    npad = -(-(n + 1) // _CH) * _CH
    rps = npad // _NS
    totch0 = -(-e // _CH)
    cpw = -(-totch0 // _NW)
    if cpw % 2:
        cpw += 1
    totch = cpw * _NW
    epad = totch * _CH

    src = edge_index[0]
    dst = edge_index[1]
    padi = jnp.full((epad - e,), n, jnp.int32)
    src2 = jnp.concatenate([src, padi]).reshape(totch, _CH)
    dst2 = jnp.concatenate([dst, padi]).reshape(totch, _CH)
    xp = jnp.pad(x, ((0, npad - n), (0, 0)))
    batp = jnp.pad(batch, (0, npad - n), constant_values=_G).reshape(npad, 1)
    zeros16 = jnp.zeros((npad, _F), jnp.float32)
    ones16 = jnp.ones((npad, _F), jnp.float32)

    agg_call = _make_agg(npad, cpw, rps)

    degc = agg_call(ones16, src2, dst2, zeros16)[:, :, 0:1]

    nblk = 16
    b = npad // nblk
    fullspec = lambda shp: pl.BlockSpec(shp, lambda i: tuple(0 for _ in shp))
    rowspec = lambda w: pl.BlockSpec((b, w), lambda i: (i, 0))
    aggspec = pl.BlockSpec((2, b, 1), lambda i: (0, i, 0))
    agg16spec = pl.BlockSpec((2, b, _F), lambda i: (0, i, 0))
    nf_out = jax.ShapeDtypeStruct((npad, _F), jnp.float32)
    n1_out = jax.ShapeDtypeStruct((npad, 1), jnp.float32)

    z, u, dis, dinv = pl.pallas_call(
        _pre_body,
        grid=(nblk,),
        in_specs=[rowspec(fin), fullspec((fin, _F)), aggspec],
        out_specs=[rowspec(_F), rowspec(_F), rowspec(1), rowspec(1)],
        out_shape=[nf_out, nf_out, n1_out, n1_out],
    )(xp, W1, degc)

    mid_call = pl.pallas_call(
        _mid_body,
        grid=(nblk,),
        in_specs=[agg16spec, rowspec(_F), rowspec(1), rowspec(1),
                  fullspec((1, _F)), fullspec((_F, _F))],
        out_specs=[rowspec(_F), rowspec(_F)],
        out_shape=[nf_out, nf_out],
    )

    for wk, bk in ((W2, b1), (W3, b2), (W4, b3), (W5, b4)):
        agg = agg_call(u, src2, dst2, zeros16)
        z, u = mid_call(agg, z, dis, dinv, bk.reshape(1, _F), wk)

    agg = agg_call(u, src2, dst2, zeros16)
    out = pl.pallas_call(
        functools.partial(_post_body, nblk=nblk),
        grid=(nblk,),
        in_specs=[agg16spec, rowspec(_F), rowspec(1), rowspec(1),
                  fullspec((1, _F)), rowspec(1), fullspec((_F, 2)),
                  fullspec((1, 2))],
        out_specs=pl.BlockSpec((_G, 2), lambda i: (0, 0)),
        out_shape=jax.ShapeDtypeStruct((_G, 2), jnp.float32),
        scratch_shapes=[pltpu.VMEM((_G, _F), jnp.float32),
                        pltpu.VMEM((_G, 1), jnp.float32)],
    )(agg, z, dis, dinv, b5.reshape(1, _F), batp, Wl, bl.reshape(1, 2))
    return out


# speedup vs baseline: 46.8790x; 2.2810x over previous
import functools

import jax
import jax.numpy as jnp
from jax import lax
from jax.experimental import pallas as pl
from jax.experimental.pallas import tpu as pltpu
from jax.experimental.pallas import tpu_sc as plsc

_NC = 2
_NS = 16
_NW = _NC * _NS
_CH = 128
_F = 16
_G = 64


_IB = 8


def _sc_agg_body(u_hbm, src_hbm, dst_hbm, zro_hbm, out_hbm,
                 sidx, didx, rows, gsem, ssem, isem, acc, *, cpw, rps, totch):
    c = lax.axis_index("c")
    s = lax.axis_index("s")
    w = c * _NS + s
    base = w * cpw
    for p in range(2):
        r0 = base + p * _IB
        pltpu.async_copy(src_hbm.at[pl.ds(r0, _IB)], sidx.at[p], isem.at[p])
        pltpu.async_copy(dst_hbm.at[pl.ds(r0, _IB)], didx.at[p], isem.at[p])
    pltpu.sync_copy(zro_hbm.at[pl.ds(s * rps, rps)], acc.at[pl.ds(s * rps, rps)])
    plsc.subcore_barrier()

    def body(k, carry):
        for p in range(2):
            sb = 2 * k + p
            row0 = base + sb * _IB
            pltpu.make_async_copy(src_hbm.at[pl.ds(row0, _IB)], sidx.at[p],
                                  isem.at[p]).wait()
            pltpu.make_async_copy(dst_hbm.at[pl.ds(row0, _IB)], didx.at[p],
                                  isem.at[p]).wait()
            for j in range(_IB):
                pltpu.async_copy(u_hbm.at[sidx.at[p, j]], rows.at[j], gsem.at[j])
            for j in range(_IB):
                pltpu.make_async_copy(u_hbm.at[sidx.at[p, j]], rows.at[j],
                                      gsem.at[j]).wait()
                pltpu.async_copy(rows.at[j], acc.at[didx.at[p, j]], ssem.at[j],
                                 add=True)
            for j in range(_IB):
                pltpu.make_async_copy(rows.at[j], acc.at[didx.at[p, j]],
                                      ssem.at[j]).wait()
            nrow = jnp.minimum(row0 + 2 * _IB, totch - _IB)
            pltpu.async_copy(src_hbm.at[pl.ds(nrow, _IB)], sidx.at[p], isem.at[p])
            pltpu.async_copy(dst_hbm.at[pl.ds(nrow, _IB)], didx.at[p], isem.at[p])
        return carry

    lax.fori_loop(0, cpw // (2 * _IB), body, 0, unroll=False)
    for p in range(2):
        pltpu.make_async_copy(src_hbm.at[pl.ds(base, _IB)], sidx.at[p],
                              isem.at[p]).wait()
        pltpu.make_async_copy(dst_hbm.at[pl.ds(base, _IB)], didx.at[p],
                              isem.at[p]).wait()
    plsc.subcore_barrier()
    pltpu.sync_copy(acc.at[pl.ds(s * rps, rps)], out_hbm.at[c, pl.ds(s * rps, rps)])


def _make_agg(npad, cpw, rps, totch):
    mesh = plsc.VectorSubcoreMesh(core_axis_name="c", subcore_axis_name="s")
    return pl.kernel(
        functools.partial(_sc_agg_body, cpw=cpw, rps=rps, totch=totch),
        out_type=jax.ShapeDtypeStruct((_NC, npad, _F), jnp.float32),
        mesh=mesh,
        scratch_types=[
            pltpu.VMEM((2, _IB, _CH), jnp.int32),
            pltpu.VMEM((2, _IB, _CH), jnp.int32),
            pltpu.VMEM((_IB, _CH, _F), jnp.float32),
            pltpu.SemaphoreType.DMA((_IB,)),
            pltpu.SemaphoreType.DMA((_IB,)),
            pltpu.SemaphoreType.DMA((2,)),
            pltpu.VMEM_SHARED((npad, _F), jnp.float32),
        ],
        compiler_params=pltpu.CompilerParams(use_tc_tiling_on_sc=False),
    )


def _sc_deg_body(dst_hbm, zro1_hbm, one_hbm, out_hbm,
                 didx, ones_v, ssem, isem, acc1, *, cpw, rps1, totch):
    c = lax.axis_index("c")
    s = lax.axis_index("s")
    w = c * _NS + s
    base = w * cpw
    for p in range(2):
        pltpu.async_copy(dst_hbm.at[pl.ds(base + p * _IB, _IB)], didx.at[p],
                         isem.at[p])
    pltpu.sync_copy(one_hbm, ones_v)
    pltpu.sync_copy(zro1_hbm.at[pl.ds(s * rps1, rps1)],
                    acc1.at[pl.ds(s * rps1, rps1)])
    plsc.subcore_barrier()

    def body(k, carry):
        for p in range(2):
            sb = 2 * k + p
            row0 = base + sb * _IB
            pltpu.make_async_copy(dst_hbm.at[pl.ds(row0, _IB)], didx.at[p],
                                  isem.at[p]).wait()
            for j in range(_IB):
                pltpu.async_copy(ones_v, acc1.at[didx.at[p, j]], ssem.at[j],
                                 add=True)
            for j in range(_IB):
                pltpu.make_async_copy(ones_v, acc1.at[didx.at[p, j]],
                                      ssem.at[j]).wait()
            nrow = jnp.minimum(row0 + 2 * _IB, totch - _IB)
            pltpu.async_copy(dst_hbm.at[pl.ds(nrow, _IB)], didx.at[p], isem.at[p])
        return carry

    lax.fori_loop(0, cpw // (2 * _IB), body, 0, unroll=False)
    for p in range(2):
        pltpu.make_async_copy(dst_hbm.at[pl.ds(base, _IB)], didx.at[p],
                              isem.at[p]).wait()
    plsc.subcore_barrier()
    pltpu.sync_copy(acc1.at[pl.ds(s * rps1, rps1)],
                    out_hbm.at[c, pl.ds(s * rps1, rps1)])


def _make_deg(npad, cpw, rps1, totch):
    mesh = plsc.VectorSubcoreMesh(core_axis_name="c", subcore_axis_name="s")
    return pl.kernel(
        functools.partial(_sc_deg_body, cpw=cpw, rps1=rps1, totch=totch),
        out_type=jax.ShapeDtypeStruct((_NC, npad), jnp.float32),
        mesh=mesh,
        scratch_types=[
            pltpu.VMEM((2, _IB, _CH), jnp.int32),
            pltpu.VMEM((_CH,), jnp.float32),
            pltpu.SemaphoreType.DMA((_IB,)),
            pltpu.SemaphoreType.DMA((2,)),
            pltpu.VMEM_SHARED((npad,), jnp.float32),
        ],
        compiler_params=pltpu.CompilerParams(use_tc_tiling_on_sc=False),
    )


def _pre_body(xr, wr, dcr, zr, ur, disr, dinvr):
    dc = dcr[...]
    deg = dc[0] + dc[1] + 1.0
    dis = lax.rsqrt(deg)
    dinv = 1.0 / deg
    z = jnp.dot(xr[...], wr[...], preferred_element_type=jnp.float32)
    zr[...] = z
    ur[...] = z * dis
    disr[...] = dis
    dinvr[...] = dinv


def _mid_body(ar, zr, disr, dinvr, br, wr, z2r, u2r):
    a = ar[...]
    dis = disr[...]
    h = jnp.maximum(dis * (a[0] + a[1]) + dinvr[...] * zr[...] + br[...], 0.0)
    z2 = jnp.dot(h, wr[...], preferred_element_type=jnp.float32)
    z2r[...] = z2
    u2r[...] = z2 * dis


def _post_body(ar, zr, disr, dinvr, br, batr, wlr, blr, outr, sacc, cacc, *, nblk):
    i = pl.program_id(0)

    @pl.when(i == 0)
    def _init():
        sacc[...] = jnp.zeros_like(sacc)
        cacc[...] = jnp.zeros_like(cacc)

    a = ar[...]
    h = jnp.maximum(disr[...] * (a[0] + a[1]) + dinvr[...] * zr[...] + br[...], 0.0)
    oh = (batr[...] == lax.broadcasted_iota(jnp.int32, (h.shape[0], _G), 1))
    oh = oh.astype(jnp.float32)
    dn = (((0,), (0,)), ((), ()))
    sacc[...] += lax.dot_general(oh, h, dn, preferred_element_type=jnp.float32)
    ones = jnp.ones((h.shape[0], 1), jnp.float32)
    cacc[...] += lax.dot_general(oh, ones, dn, preferred_element_type=jnp.float32)

    @pl.when(i == nblk - 1)
    def _fin():
        mean = sacc[...] / jnp.maximum(cacc[...], 1.0)
        logits = jnp.dot(mean, wlr[...], preferred_element_type=jnp.float32) + blr[...]
        m = jnp.max(logits, axis=1, keepdims=True)
        lse = jnp.log(jnp.sum(jnp.exp(logits - m), axis=1, keepdims=True)) + m
        outr[...] = logits - lse


def kernel(x, edge_index, batch, W1, b1, W2, b2, W3, b3, W4, b4, W5, b5, Wl, bl):
    n, fin = x.shape
    e = edge_index.shape[1]
    npad = -(-(n + 1) // _CH) * _CH
    rps = npad // _NS
    totch0 = -(-e // _CH)
    cpw = -(-totch0 // _NW)
    cpw = -(-cpw // (2 * _IB)) * (2 * _IB)
    totch = cpw * _NW
    epad = totch * _CH

    src = edge_index[0]
    dst = edge_index[1]
    padi = jnp.full((epad - e,), n, jnp.int32)
    src2 = jnp.concatenate([src, padi]).reshape(totch, _CH)
    dst2 = jnp.concatenate([dst, padi]).reshape(totch, _CH)
    xp = jnp.pad(x, ((0, npad - n), (0, 0)))
    batp = jnp.pad(batch, (0, npad - n), constant_values=_G).reshape(npad, 1)
    zeros16 = jnp.zeros((npad, _F), jnp.float32)
    zeros1 = jnp.zeros((npad,), jnp.float32)
    ones128 = jnp.ones((_CH,), jnp.float32)

    agg_call = _make_agg(npad, cpw, rps, totch)
    deg_call = _make_deg(npad, cpw, rps, totch)

    degc = deg_call(dst2, zeros1, ones128).reshape(_NC, npad, 1)

    nblk = 16
    b = npad // nblk
    fullspec = lambda shp: pl.BlockSpec(shp, lambda i: tuple(0 for _ in shp))
    rowspec = lambda w: pl.BlockSpec((b, w), lambda i: (i, 0))
    aggspec = pl.BlockSpec((2, b, 1), lambda i: (0, i, 0))
    agg16spec = pl.BlockSpec((2, b, _F), lambda i: (0, i, 0))
    nf_out = jax.ShapeDtypeStruct((npad, _F), jnp.float32)
    n1_out = jax.ShapeDtypeStruct((npad, 1), jnp.float32)

    z, u, dis, dinv = pl.pallas_call(
        _pre_body,
        grid=(nblk,),
        in_specs=[rowspec(fin), fullspec((fin, _F)), aggspec],
        out_specs=[rowspec(_F), rowspec(_F), rowspec(1), rowspec(1)],
        out_shape=[nf_out, nf_out, n1_out, n1_out],
    )(xp, W1, degc)

    mid_call = pl.pallas_call(
        _mid_body,
        grid=(nblk,),
        in_specs=[agg16spec, rowspec(_F), rowspec(1), rowspec(1),
                  fullspec((1, _F)), fullspec((_F, _F))],
        out_specs=[rowspec(_F), rowspec(_F)],
        out_shape=[nf_out, nf_out],
    )

    for wk, bk in ((W2, b1), (W3, b2), (W4, b3), (W5, b4)):
        agg = agg_call(u, src2, dst2, zeros16)
        z, u = mid_call(agg, z, dis, dinv, bk.reshape(1, _F), wk)

    agg = agg_call(u, src2, dst2, zeros16)
    out = pl.pallas_call(
        functools.partial(_post_body, nblk=nblk),
        grid=(nblk,),
        in_specs=[agg16spec, rowspec(_F), rowspec(1), rowspec(1),
                  fullspec((1, _F)), rowspec(1), fullspec((_F, 2)),
                  fullspec((1, 2))],
        out_specs=pl.BlockSpec((_G, 2), lambda i: (0, 0)),
        out_shape=jax.ShapeDtypeStruct((_G, 2), jnp.float32),
        scratch_shapes=[pltpu.VMEM((_G, _F), jnp.float32),
                        pltpu.VMEM((_G, 1), jnp.float32)],
    )(agg, z, dis, dinv, b5.reshape(1, _F), batp, Wl, bl.reshape(1, 2))
    return out


# speedup vs baseline: 64.6029x; 1.3781x over previous
import functools

import jax
import jax.numpy as jnp
from jax import lax
from jax.experimental import pallas as pl
from jax.experimental.pallas import tpu as pltpu
from jax.experimental.pallas import tpu_sc as plsc

_NC = 2
_NS = 16
_NW = _NC * _NS
_CH = 128
_F = 16
_G = 64


_IB = 8


def _sc_agg_body(u_hbm, src_hbm, dst_hbm, zro_hbm, out_hbm,
                 sidx, didx, rows, gsem, ssem, isem, acc, *, cpw, rps, totch):
    c = lax.axis_index("c")
    s = lax.axis_index("s")
    w = c * _NS + s
    base = w * cpw
    for p in range(2):
        r0 = base + p * _IB
        pltpu.async_copy(src_hbm.at[pl.ds(r0, _IB)], sidx.at[p], isem.at[p])
        pltpu.async_copy(dst_hbm.at[pl.ds(r0, _IB)], didx.at[p], isem.at[p])
    pltpu.sync_copy(zro_hbm.at[pl.ds(s * rps, rps)], acc.at[pl.ds(s * rps, rps)])
    plsc.subcore_barrier()

    def body(k, carry):
        for p in range(2):
            sb = 2 * k + p
            row0 = base + sb * _IB
            pltpu.make_async_copy(src_hbm.at[pl.ds(row0, _IB)], sidx.at[p],
                                  isem.at[p]).wait()
            pltpu.make_async_copy(dst_hbm.at[pl.ds(row0, _IB)], didx.at[p],
                                  isem.at[p]).wait()
            for j in range(_IB):
                pltpu.async_copy(u_hbm.at[sidx.at[p, j]], rows.at[j], gsem.at[j])
            for j in range(_IB):
                pltpu.make_async_copy(u_hbm.at[sidx.at[p, j]], rows.at[j],
                                      gsem.at[j]).wait()
                pltpu.async_copy(rows.at[j], acc.at[didx.at[p, j]], ssem.at[j],
                                 add=True)
            for j in range(_IB):
                pltpu.make_async_copy(rows.at[j], acc.at[didx.at[p, j]],
                                      ssem.at[j]).wait()
            nrow = jnp.minimum(row0 + 2 * _IB, totch - _IB)
            pltpu.async_copy(src_hbm.at[pl.ds(nrow, _IB)], sidx.at[p], isem.at[p])
            pltpu.async_copy(dst_hbm.at[pl.ds(nrow, _IB)], didx.at[p], isem.at[p])
        return carry

    lax.fori_loop(0, cpw // (2 * _IB), body, 0, unroll=False)
    for p in range(2):
        pltpu.make_async_copy(src_hbm.at[pl.ds(base, _IB)], sidx.at[p],
                              isem.at[p]).wait()
        pltpu.make_async_copy(dst_hbm.at[pl.ds(base, _IB)], didx.at[p],
                              isem.at[p]).wait()
    plsc.subcore_barrier()
    pltpu.sync_copy(acc.at[pl.ds(s * rps, rps)], out_hbm.at[c, pl.ds(s * rps, rps)])


def _make_agg(npad, cpw, rps, totch):
    mesh = plsc.VectorSubcoreMesh(core_axis_name="c", subcore_axis_name="s")
    return pl.kernel(
        functools.partial(_sc_agg_body, cpw=cpw, rps=rps, totch=totch),
        out_type=jax.ShapeDtypeStruct((_NC, npad, _F), jnp.float32),
        mesh=mesh,
        scratch_types=[
            pltpu.VMEM((2, _IB, _CH), jnp.int32),
            pltpu.VMEM((2, _IB, _CH), jnp.int32),
            pltpu.VMEM((_IB, _CH, _F), jnp.float32),
            pltpu.SemaphoreType.DMA((_IB,)),
            pltpu.SemaphoreType.DMA((_IB,)),
            pltpu.SemaphoreType.DMA((2,)),
            pltpu.VMEM_SHARED((npad, _F), jnp.float32),
        ],
        compiler_params=pltpu.CompilerParams(use_tc_tiling_on_sc=False),
    )


def _sc_deg_body(dst_hbm, zro1_hbm, one_hbm, out_hbm,
                 didx, ones_v, ssem, isem, acc1, *, cpw, rps1, totch):
    c = lax.axis_index("c")
    s = lax.axis_index("s")
    w = c * _NS + s
    base = w * cpw
    for p in range(2):
        pltpu.async_copy(dst_hbm.at[pl.ds(base + p * _IB, _IB)], didx.at[p],
                         isem.at[p])
    pltpu.sync_copy(one_hbm, ones_v)
    pltpu.sync_copy(zro1_hbm.at[pl.ds(s * rps1, rps1)],
                    acc1.at[pl.ds(s * rps1, rps1)])
    plsc.subcore_barrier()

    def body(k, carry):
        for p in range(2):
            sb = 2 * k + p
            row0 = base + sb * _IB
            pltpu.make_async_copy(dst_hbm.at[pl.ds(row0, _IB)], didx.at[p],
                                  isem.at[p]).wait()
            for j in range(_IB):
                pltpu.async_copy(ones_v, acc1.at[didx.at[p, j]], ssem.at[j],
                                 add=True)
            for j in range(_IB):
                pltpu.make_async_copy(ones_v, acc1.at[didx.at[p, j]],
                                      ssem.at[j]).wait()
            nrow = jnp.minimum(row0 + 2 * _IB, totch - _IB)
            pltpu.async_copy(dst_hbm.at[pl.ds(nrow, _IB)], didx.at[p], isem.at[p])
        return carry

    lax.fori_loop(0, cpw // (2 * _IB), body, 0, unroll=False)
    for p in range(2):
        pltpu.make_async_copy(dst_hbm.at[pl.ds(base, _IB)], didx.at[p],
                              isem.at[p]).wait()
    plsc.subcore_barrier()
    pltpu.sync_copy(acc1.at[pl.ds(s * rps1, rps1)],
                    out_hbm.at[c, pl.ds(s * rps1, rps1)])


def _make_deg(npad, cpw, rps1, totch):
    mesh = plsc.VectorSubcoreMesh(core_axis_name="c", subcore_axis_name="s")
    return pl.kernel(
        functools.partial(_sc_deg_body, cpw=cpw, rps1=rps1, totch=totch),
        out_type=jax.ShapeDtypeStruct((_NC, npad), jnp.float32),
        mesh=mesh,
        scratch_types=[
            pltpu.VMEM((2, _IB, _CH), jnp.int32),
            pltpu.VMEM((_CH,), jnp.float32),
            pltpu.SemaphoreType.DMA((_IB,)),
            pltpu.SemaphoreType.DMA((2,)),
            pltpu.VMEM_SHARED((npad,), jnp.float32),
        ],
        compiler_params=pltpu.CompilerParams(use_tc_tiling_on_sc=False),
    )


def _pre_body_r2(xr, wr, dcr, zr, ur, disr, dinvr):
    dc = dcr[...]
    deg = dc[0] + dc[1] + 1.0
    dis = lax.rsqrt(deg)
    dinv = 1.0 / deg
    z = jnp.dot(xr[...], wr[...], preferred_element_type=jnp.float32)
    zr[...] = z
    ur[...] = z * dis
    disr[...] = dis
    dinvr[...] = dinv


def _pre_body(xr, wr, dcr, spr, zr, ur, disr, dinvr):
    dc = dcr[...]
    deg8 = dc[0] + dc[1] + 1.0
    deg = jnp.dot(deg8, spr[...], preferred_element_type=jnp.float32)
    dis = lax.rsqrt(deg)
    dinv = 1.0 / deg
    z = jnp.dot(xr[...], wr[...], preferred_element_type=jnp.float32)
    zr[...] = z
    ur[...] = z * dis
    disr[...] = dis
    dinvr[...] = dinv


def _mid_body(ar, zr, disr, dinvr, br, wr, z2r, u2r):
    a = ar[...]
    dis = disr[...]
    h = jnp.maximum(dis * (a[0] + a[1]) + dinvr[...] * zr[...] + br[...], 0.0)
    z2 = jnp.dot(h, wr[...], preferred_element_type=jnp.float32)
    z2r[...] = z2
    u2r[...] = z2 * dis


def _post_body(ar, zr, disr, dinvr, br, batr, wlr, blr, outr, sacc, cacc, *, nblk):
    i = pl.program_id(0)

    @pl.when(i == 0)
    def _init():
        sacc[...] = jnp.zeros_like(sacc)
        cacc[...] = jnp.zeros_like(cacc)

    a = ar[...]
    h = jnp.maximum(disr[...] * (a[0] + a[1]) + dinvr[...] * zr[...] + br[...], 0.0)
    oh = (batr[...] == lax.broadcasted_iota(jnp.int32, (h.shape[0], _G), 1))
    oh = oh.astype(jnp.float32)
    dn = (((0,), (0,)), ((), ()))
    sacc[...] += lax.dot_general(oh, h, dn, preferred_element_type=jnp.float32)
    ones = jnp.ones((h.shape[0], 1), jnp.float32)
    cacc[...] += lax.dot_general(oh, ones, dn, preferred_element_type=jnp.float32)

    @pl.when(i == nblk - 1)
    def _fin():
        mean = sacc[...] / jnp.maximum(cacc[...], 1.0)
        logits = jnp.dot(mean, wlr[...], preferred_element_type=jnp.float32) + blr[...]
        m = jnp.max(logits, axis=1, keepdims=True)
        lse = jnp.log(jnp.sum(jnp.exp(logits - m), axis=1, keepdims=True)) + m
        outr[...] = logits - lse


def kernel(x, edge_index, batch, W1, b1, W2, b2, W3, b3, W4, b4, W5, b5, Wl, bl):
    n, fin = x.shape
    e = edge_index.shape[1]
    npad = -(-(n + 1) // _CH) * _CH
    rps = npad // _NS
    totch0 = -(-e // _CH)
    cpw = -(-totch0 // _NW)
    cpw = -(-cpw // (2 * _IB)) * (2 * _IB)
    totch = cpw * _NW
    epad = totch * _CH

    src = edge_index[0]
    dst = edge_index[1]
    padi = jnp.full((epad - e,), n, jnp.int32)
    src2 = jnp.concatenate([src, padi]).reshape(totch, _CH)
    dst2 = jnp.concatenate([dst, padi]).reshape(totch, _CH)
    xp = jnp.pad(x, ((0, npad - n), (0, 0)))
    batp = jnp.pad(batch, (0, npad - n), constant_values=_G).reshape(npad, 1)
    zeros16 = jnp.zeros((npad, _F), jnp.float32)
    zeros1 = jnp.zeros((npad,), jnp.float32)
    ones128 = jnp.ones((_CH,), jnp.float32)

    agg_call = _make_agg(npad, cpw, rps, totch)
    deg_call = _make_deg(npad, cpw, rps, totch)

    degc = deg_call(dst2, zeros1, ones128)

    r8 = npad // 8
    nblk = 4
    rb = r8 // nblk

    def big(w):
        return jnp.kron(jnp.eye(8, dtype=jnp.float32), w)

    fullspec = lambda shp: pl.BlockSpec(shp, lambda i: tuple(0 for _ in shp))
    rowspec = lambda w: pl.BlockSpec((rb, w), lambda i: (i, 0))
    degspec = pl.BlockSpec((2, rb, 8), lambda i: (0, i, 0))
    aggpspec = pl.BlockSpec((2, rb, 128), lambda i: (0, i, 0))
    pk_out = jax.ShapeDtypeStruct((r8, 128), jnp.float32)

    nblkp0 = 16
    b0 = npad // nblkp0
    rs0 = lambda w: pl.BlockSpec((b0, w), lambda i: (i, 0))
    z, u, dis, dinv = pl.pallas_call(
        _pre_body_r2,
        grid=(nblkp0,),
        in_specs=[rs0(fin), fullspec((fin, _F)),
                  pl.BlockSpec((2, b0, 1), lambda i: (0, i, 0))],
        out_specs=[rs0(_F), rs0(_F), rs0(1), rs0(1)],
        out_shape=[jax.ShapeDtypeStruct((npad, _F), jnp.float32),
                   jax.ShapeDtypeStruct((npad, _F), jnp.float32),
                   jax.ShapeDtypeStruct((npad, 1), jnp.float32),
                   jax.ShapeDtypeStruct((npad, 1), jnp.float32)],
    )(xp, W1, degc.reshape(_NC, npad, 1))
    z = z.reshape(r8, 128)
    u = u.reshape(r8, 128)
    dis = jnp.broadcast_to(dis, (npad, _F)).reshape(r8, 128)
    dinv = jnp.broadcast_to(dinv, (npad, _F)).reshape(r8, 128)

    mid_call = pl.pallas_call(
        _mid_body,
        grid=(nblk,),
        in_specs=[aggpspec, rowspec(128), rowspec(128), rowspec(128),
                  fullspec((1, 128)), fullspec((128, 128))],
        out_specs=[rowspec(128), rowspec(128)],
        out_shape=[pk_out, pk_out],
    )

    for wk, bk in ((W2, b1), (W3, b2), (W4, b3), (W5, b4)):
        agg = agg_call(u.reshape(npad, _F), src2, dst2, zeros16)
        z, u = mid_call(agg.reshape(_NC, r8, 128), z, dis, dinv,
                        jnp.tile(bk, 8).reshape(1, 128), big(wk))

    agg = agg_call(u.reshape(npad, _F), src2, dst2, zeros16)
    nblkp = 16
    bu = npad // nblkp
    urowspec = lambda w: pl.BlockSpec((bu, w), lambda i: (i, 0))
    agg16spec = pl.BlockSpec((2, bu, _F), lambda i: (0, i, 0))
    out = pl.pallas_call(
        functools.partial(_post_body, nblk=nblkp),
        grid=(nblk,),
        in_specs=[agg16spec, urowspec(_F), urowspec(_F), urowspec(_F),
                  fullspec((1, _F)), urowspec(1), fullspec((_F, 2)),
                  fullspec((1, 2))],
        out_specs=pl.BlockSpec((_G, 2), lambda i: (0, 0)),
        out_shape=jax.ShapeDtypeStruct((_G, 2), jnp.float32),
        scratch_shapes=[pltpu.VMEM((_G, _F), jnp.float32),
                        pltpu.VMEM((_G, 1), jnp.float32)],
    )(agg, z.reshape(npad, _F), dis.reshape(npad, _F), dinv.reshape(npad, _F),
      b5.reshape(1, _F), batp, Wl, bl.reshape(1, 2))
    return out
